# baseline (speedup 1.0000x reference)
import functools

import jax
import jax.numpy as jnp
from jax import lax
from jax.experimental import pallas as pl
from jax.experimental.pallas import tpu as pltpu
from jax.experimental.pallas import tpu_sc as plsc

EMBED_DIM = 64
NUM_ROWS = 16384
ROW_LEN = 50
NUM_CORES = 2
NUM_SUBCORES = 16
NUM_WORKERS = NUM_CORES * NUM_SUBCORES

TOTAL = NUM_ROWS * ROW_LEN
IBLK = 128
BLOCKS = ROW_LEN * (NUM_ROWS // IBLK)
BLOCKS_PER_WORKER = BLOCKS // NUM_WORKERS
_TRV = (IBLK * EMBED_DIM) // 16

_mesh = plsc.VectorSubcoreMesh(core_axis_name="c", subcore_axis_name="s")


@functools.partial(
    pl.kernel,
    mesh=_mesh,
    out_type=jax.ShapeDtypeStruct((TOTAL,), jnp.int32),
    scratch_types=[
        pltpu.VMEM((NUM_ROWS,), jnp.int32),
        pltpu.VMEM((NUM_ROWS,), jnp.int32),
    ],
)
def _detile_kernel(idxT_hbm, out_hbm, row_v0, row_v1):
    wid = lax.axis_index("s") * NUM_CORES + lax.axis_index("c")
    rows_v = [row_v0, row_v1]
    for rep in range(2):
        j = wid + NUM_WORKERS * rep
        @pl.when(j < ROW_LEN)
        def _():
            pltpu.sync_copy(idxT_hbm.at[j], rows_v[rep])
            pltpu.sync_copy(rows_v[rep],
                            out_hbm.at[pl.ds(j * NUM_ROWS, NUM_ROWS)])


@functools.partial(
    pl.kernel,
    mesh=_mesh,
    out_type=jax.ShapeDtypeStruct((ROW_LEN, EMBED_DIM, NUM_ROWS), jnp.float32),
    scratch_types=[
        pltpu.VMEM((2, IBLK), jnp.int32),
        pltpu.VMEM((2, IBLK, 2 * EMBED_DIM), jnp.float32),
        pltpu.VMEM((EMBED_DIM, IBLK), jnp.float32),
        pltpu.SemaphoreType.DMA,
        pltpu.SemaphoreType.DMA,
        pltpu.SemaphoreType.DMA,
        pltpu.SemaphoreType.DMA,
    ],
    compiler_params=pltpu.CompilerParams(needs_layout_passes=False),
)
def _gather_kernel(idx_hbm, table_hbm, out_hbm, idx_v, rows_v, tr_v,
                   si0, si1, sg0, sg1):
    sems_i = [si0, si1]
    sems_g = [sg0, sg1]
    wid = lax.axis_index("s") * NUM_CORES + lax.axis_index("c")
    blk0 = wid * BLOCKS_PER_WORKER
    lane = jnp.arange(16, dtype=jnp.int32)

    def idx_off(t):
        blk = blk0 + t
        j = blk // (NUM_ROWS // IBLK)
        i0 = (blk % (NUM_ROWS // IBLK)) * IBLK
        return j, i0, j * NUM_ROWS + i0

    def start_idx(t, b):
        _, _, off = idx_off(t)
        pltpu.async_copy(idx_hbm.at[pl.ds(off, IBLK)], idx_v.at[b], sems_i[b])

    def wait_idx(t, b):
        _, _, off = idx_off(t)
        pltpu.make_async_copy(idx_hbm.at[pl.ds(off, IBLK)], idx_v.at[b],
                              sems_i[b]).wait()

    def start_gather(b):
        pltpu.async_copy(table_hbm.at[idx_v.at[b]], rows_v.at[b], sems_g[b])

    def wait_gather(b):
        pltpu.make_async_copy(table_hbm.at[idx_v.at[b]], rows_v.at[b],
                              sems_g[b]).wait()

    def transpose_and_store(t, b):
        j, i0, _ = idx_off(t)
        src = rows_v.at[b]

        def outer(kq, _):
            kv = kq + jnp.zeros((16,), jnp.int32)
            for u in range(IBLK // 16):
                vals = plsc.load_gather(src, [u * 16 + lane, kv])
                tr_v.at[kq][pl.ds(u * 16, 16)] = vals
            return 0

        lax.fori_loop(0, EMBED_DIM, outer, 0)
        pltpu.sync_copy(tr_v, out_hbm.at[j].at[:, pl.ds(i0, IBLK)])

    pltpu.sync_copy(idx_hbm.at[pl.ds(idx_off(0)[2], IBLK)], idx_v.at[0])
    start_gather(0)
    start_idx(1, 1)

    def step(t, b):
        nb = 1 - b
        wait_gather(b)
        wait_idx(t + 1, nb)
        start_gather(nb)
        start_idx(t + 2, b)
        transpose_and_store(t, b)

    def pair(p, _):
        t = p * 2
        step(t, 0)
        step(t + 1, 1)
        return 0

    lax.fori_loop(0, (BLOCKS_PER_WORKER - 2) // 2, pair, 0)

    t = BLOCKS_PER_WORKER - 2
    b = t % 2
    wait_gather(b)
    wait_idx(t + 1, (t + 1) % 2)
    start_gather((t + 1) % 2)
    transpose_and_store(t, b)
    t = BLOCKS_PER_WORKER - 1
    b = t % 2
    wait_gather(b)
    transpose_and_store(t, b)


def kernel(input, table):
    flat_idx = _detile_kernel(input.astype(jnp.int32).T)
    table_padded = jnp.pad(table, ((0, 0), (0, EMBED_DIM)))
    out = _gather_kernel(flat_idx, table_padded)
    return out.transpose(2, 0, 1)
